# Initial kernel scaffold; baseline (speedup 1.0000x reference)
import jax
import jax.numpy as jnp
from jax.experimental import pallas as pl


def kernel(x, W):
    raise NotImplementedError("write your pallas kernel here")


# speedup vs baseline: 4.3845x; 4.3845x over previous
import functools

import jax
import jax.numpy as jnp
from jax import lax
from jax.experimental import pallas as pl
from jax.experimental.pallas import tpu as pltpu
from jax.experimental.pallas import tpu_sc as plsc

_N_FIELDS = 26
_VOCAB = 1000
_EMB = 128
_BATCH = 16384

_ROWS = _BATCH * _N_FIELDS
_NC = 2
_NS = 16
_NW = _NC * _NS
_RPW = _ROWS // _NW
_CHUNK = 128
_NCH = _RPW // _CHUNK
_NBUF = 4
_NG = _NCH // _NBUF


def _sc_body(x_hbm, tab_hbm, out_hbm, xv, idxv, b0, b1, b2, b3,
             g0, g1, g2, g3, w0, w1, w2, w3):
    bufs = (b0, b1, b2, b3)
    gsem = (g0, g1, g2, g3)
    wsem = (w0, w1, w2, w3)

    wid = lax.axis_index("s") * _NC + lax.axis_index("c")
    base = pl.multiple_of(wid * _RPW, _RPW)

    pltpu.sync_copy(x_hbm.at[pl.ds(base, _RPW)], xv)

    lane = lax.iota(jnp.int32, 16)

    def fill(j, carry):
        for c in range(8):
            p0 = pl.multiple_of(j * _CHUNK + c * 16, 16)
            v = xv[pl.ds(p0, 16)]
            f = lax.rem(p0 + lane, jnp.int32(_N_FIELDS))
            idxv[j, pl.ds(c * 16, 16)] = v + f * jnp.int32(_VOCAB)
        return carry

    lax.fori_loop(0, _NCH, fill, 0)

    def start_gather(c, b):
        pltpu.async_copy(tab_hbm.at[idxv.at[c]], bufs[b], gsem[b])

    def wait_gather(b):
        pltpu.make_async_copy(tab_hbm.at[idxv.at[0]], bufs[b], gsem[b]).wait()

    def start_write(c, b):
        pltpu.async_copy(bufs[b], out_hbm.at[pl.ds(base + c * _CHUNK, _CHUNK)],
                         wsem[b])

    def wait_write(b):
        pltpu.make_async_copy(bufs[b], out_hbm.at[pl.ds(base, _CHUNK)],
                              wsem[b]).wait()

    for b in range(_NBUF):
        start_gather(b, b)

    def group(t, carry):
        for b in range(_NBUF):
            c = t * _NBUF + b
            wait_gather(b)
            start_write(c, b)
        for b in range(_NBUF):
            wait_write(b)
            nc = t * _NBUF + b + _NBUF

            @pl.when(nc < _NCH)
            def _():
                start_gather(nc, b)
        return carry

    lax.fori_loop(0, _NG, group, 0)


@jax.jit
def _sc_gather(x_flat, tab):
    mesh = plsc.VectorSubcoreMesh(core_axis_name="c", subcore_axis_name="s")
    run = pl.kernel(
        _sc_body,
        mesh=mesh,
        out_type=jax.ShapeDtypeStruct((_ROWS, _EMB), jnp.float32),
        scratch_types=(
            [pltpu.VMEM((_RPW,), jnp.int32),
             pltpu.VMEM((_NCH, _CHUNK), jnp.int32)]
            + [pltpu.VMEM((_CHUNK, _EMB), jnp.float32)] * _NBUF
            + [pltpu.SemaphoreType.DMA] * (2 * _NBUF)
        ),
    )
    return run(x_flat, tab)


def kernel(x, W):
    x_flat = x.astype(jnp.int32).reshape(_ROWS)
    tab = W.reshape(_N_FIELDS * _VOCAB, _EMB)
    out = _sc_gather(x_flat, tab)
    return out.reshape(_BATCH, _N_FIELDS * _EMB)


# speedup vs baseline: 8.9616x; 2.0439x over previous
import functools

import jax
import jax.numpy as jnp
from jax import lax
from jax.experimental import pallas as pl
from jax.experimental.pallas import tpu as pltpu
from jax.experimental.pallas import tpu_sc as plsc

_N_FIELDS = 26
_VOCAB = 1000
_EMB = 128
_BATCH = 16384

_NC = 2
_NS = 16
_NW = _NC * _NS
_BPW = _BATCH // _NW
_RPW = _BPW * _N_FIELDS
_CHUNK = 128
_NQ = _BPW // _CHUNK
_NCH = _N_FIELDS * _NQ
_NBUF = 4
_NG = _NCH // _NBUF


def _sc_body(xt_hbm, tab_hbm, out_hbm, xv, idxv, b0, b1, b2, b3, ssem,
             g0, g1, g2, g3, w0, w1, w2, w3):
    bufs = (b0, b1, b2, b3)
    gsem = (g0, g1, g2, g3)
    wsem = (w0, w1, w2, w3)

    wid = lax.axis_index("s") * _NC + lax.axis_index("c")
    row0 = pl.multiple_of(wid * _BPW, _BPW)

    for i in range(_N_FIELDS):
        pltpu.async_copy(xt_hbm.at[pl.ds(i * _BATCH + row0, _BPW)],
                         xv.at[i], ssem)
    for i in range(_N_FIELDS):
        pltpu.make_async_copy(xt_hbm.at[pl.ds(0, _BPW)], xv.at[0], ssem).wait()

    def fill(c, carry):
        i = c // _NQ
        q = lax.rem(c, _NQ)
        for v in range(8):
            val = xv[i, pl.ds(q * _CHUNK + v * 16, 16)]
            idxv[c, pl.ds(v * 16, 16)] = val + i * _VOCAB
        return carry

    lax.fori_loop(0, _NCH, fill, 0)

    def out_slice(c):
        i = c // _NQ
        q = lax.rem(c, _NQ)
        return out_hbm.at[pl.ds(row0 + q * _CHUNK, _CHUNK),
                          pl.ds(i * _EMB, _EMB)]

    def start_gather(c, b):
        pltpu.async_copy(tab_hbm.at[idxv.at[c]], bufs[b], gsem[b])

    def wait_gather(b):
        pltpu.make_async_copy(tab_hbm.at[idxv.at[0]], bufs[b], gsem[b]).wait()

    def start_write(c, b):
        pltpu.async_copy(bufs[b], out_slice(c), wsem[b])

    def wait_write(b):
        pltpu.make_async_copy(bufs[b], out_slice(0), wsem[b]).wait()

    for b in range(_NBUF):
        start_gather(b, b)

    def group(t, carry):
        for b in range(_NBUF):
            c = t * _NBUF + b
            wait_gather(b)
            start_write(c, b)
        for b in range(_NBUF):
            wait_write(b)
            nc = t * _NBUF + b + _NBUF

            @pl.when(nc < _NCH)
            def _():
                start_gather(nc, b)
        return carry

    lax.fori_loop(0, _NG, group, 0)


@jax.jit
def _sc_gather(x_flat, tab):
    mesh = plsc.VectorSubcoreMesh(core_axis_name="c", subcore_axis_name="s")
    run = pl.kernel(
        _sc_body,
        mesh=mesh,
        out_type=jax.ShapeDtypeStruct((_BATCH, _N_FIELDS * _EMB), jnp.float32),
        compiler_params=pltpu.CompilerParams(use_tc_tiling_on_sc=True),
        scratch_types=(
            [pltpu.VMEM((_N_FIELDS, _BPW), jnp.int32),
             pltpu.VMEM((_NCH, _CHUNK), jnp.int32)]
            + [pltpu.VMEM((_CHUNK, _EMB), jnp.float32)] * _NBUF
            + [pltpu.SemaphoreType.DMA] * (2 * _NBUF + 1)
        ),
    )
    return run(x_flat, tab)


def kernel(x, W):
    xt = x.astype(jnp.int32).T.reshape(_BATCH * _N_FIELDS)
    tab = W.reshape(_N_FIELDS * _VOCAB, _EMB)
    return _sc_gather(xt, tab)
